# baseline (speedup 1.0000x reference)
import functools

import jax
import jax.numpy as jnp
from jax import lax
from jax.experimental import pallas as pl
from jax.experimental.pallas import tpu as pltpu
from jax.experimental.pallas import tpu_sc as plsc

N_FACTORS = 50
BATCH = 16384
NC, NS, L = 2, 16, 16
NW = NC * NS
BPW = BATCH // NW
GCH = 128
NGC = BPW // GCH


def _body(cats_hbm, u_hbm, m_hbm, out_hbm,
          cats_v, idx_u, idx_m, u_rows, m_rows, out_v, sem):
    wid = lax.axis_index("s") * NC + lax.axis_index("c")
    base = wid * BPW
    lanes = lax.iota(jnp.int32, L)

    pltpu.sync_copy(cats_hbm.at[pl.ds(base, BPW)], cats_v)
    col0 = jnp.zeros((L,), jnp.int32)
    col1 = jnp.ones((L,), jnp.int32)

    def deint_body(c, _):
        rows = c * L + lanes
        idx_u[pl.ds(c * L, L)] = plsc.load_gather(cats_v, [rows, col0])
        idx_m[pl.ds(c * L, L)] = plsc.load_gather(cats_v, [rows, col1])
        return 0

    lax.fori_loop(0, BPW // L, deint_body, 0)

    copies = []
    for g in range(NGC):
        s = pl.ds(g * GCH, GCH)
        copies.append(pltpu.async_copy(u_hbm.at[idx_u.at[s]], u_rows.at[s], sem))
        copies.append(pltpu.async_copy(m_hbm.at[idx_m.at[s]], m_rows.at[s], sem))
    for cp in copies:
        cp.wait()

    def chunk_body(c, _):
        rows = c * L + lanes
        acc = jnp.zeros((L,), jnp.float32)
        for j in range(N_FACTORS):
            col = jnp.full((L,), j, jnp.int32)
            uv = plsc.load_gather(u_rows, [rows, col])
            mv = plsc.load_gather(m_rows, [rows, col])
            acc = acc + uv * mv
        out_v[pl.ds(c * L, L)] = acc
        return 0

    lax.fori_loop(0, BPW // L, chunk_body, 0)

    pltpu.sync_copy(out_v, out_hbm.at[pl.ds(base, BPW)])


@jax.jit
def _embedding_dot(cats, u_weight, m_weight):
    mesh = plsc.VectorSubcoreMesh(core_axis_name="c", subcore_axis_name="s")
    run = pl.kernel(
        _body, mesh=mesh,
        compiler_params=pltpu.CompilerParams(
            needs_layout_passes=False, use_tc_tiling_on_sc=False),
        out_type=jax.ShapeDtypeStruct((BATCH,), jnp.float32),
        scratch_types=[
            pltpu.VMEM((BPW, 2), jnp.int32),
            pltpu.VMEM((BPW,), jnp.int32),
            pltpu.VMEM((BPW,), jnp.int32),
            pltpu.VMEM((BPW, N_FACTORS), jnp.float32),
            pltpu.VMEM((BPW, N_FACTORS), jnp.float32),
            pltpu.VMEM((BPW,), jnp.float32),
            pltpu.SemaphoreType.DMA,
        ],
    )
    return run(cats, u_weight, m_weight)


def kernel(cats, conts, u_weight, m_weight):
    del conts
    return _embedding_dot(cats.astype(jnp.int32), u_weight, m_weight)


# speedup vs baseline: 1.2939x; 1.2939x over previous
import jax
import jax.numpy as jnp
from jax import lax
from jax.experimental import pallas as pl
from jax.experimental.pallas import tpu as pltpu
from jax.experimental.pallas import tpu_sc as plsc

N_FACTORS = 50
DPAD = 128
BATCH = 16384
NC, NS = 2, 16
NW = NC * NS
BPW = BATCH // NW
CHUNK = 128


def _gather_body(iu_hbm, im_hbm, up_hbm, mp_hbm, uo_hbm, mo_hbm,
                 iu_v, im_v, u_rows, m_rows, sem):
    wid = lax.axis_index("s") * NC + lax.axis_index("c")
    base = wid * BPW

    pltpu.sync_copy(iu_hbm.at[pl.ds(base, BPW)], iu_v)
    pltpu.sync_copy(im_hbm.at[pl.ds(base, BPW)], im_v)

    def chunk(c, _):
        off = c * CHUNK
        cu = pltpu.async_copy(up_hbm.at[iu_v.at[pl.ds(off, CHUNK)]], u_rows, sem)
        cm = pltpu.async_copy(mp_hbm.at[im_v.at[pl.ds(off, CHUNK)]], m_rows, sem)
        cu.wait()
        cm.wait()
        pltpu.sync_copy(u_rows, uo_hbm.at[pl.ds(base + off, CHUNK), :])
        pltpu.sync_copy(m_rows, mo_hbm.at[pl.ds(base + off, CHUNK), :])
        return 0

    lax.fori_loop(0, BPW // CHUNK, chunk, 0)


def _dot_body(u_ref, m_ref, o_ref):
    o_ref[...] = jnp.sum(u_ref[...] * m_ref[...], axis=1)


@jax.jit
def _embedding_dot(cats, u_weight, m_weight):
    users = cats[:, 0]
    movies = cats[:, 1]
    up = jnp.pad(u_weight, ((0, 0), (0, DPAD - N_FACTORS)))
    mp = jnp.pad(m_weight, ((0, 0), (0, DPAD - N_FACTORS)))

    mesh = plsc.VectorSubcoreMesh(core_axis_name="c", subcore_axis_name="s")
    gather = pl.kernel(
        _gather_body, mesh=mesh,
        compiler_params=pltpu.CompilerParams(needs_layout_passes=False),
        out_type=[jax.ShapeDtypeStruct((BATCH, DPAD), jnp.float32),
                  jax.ShapeDtypeStruct((BATCH, DPAD), jnp.float32)],
        scratch_types=[
            pltpu.VMEM((BPW,), jnp.int32),
            pltpu.VMEM((BPW,), jnp.int32),
            pltpu.VMEM((CHUNK, DPAD), jnp.float32),
            pltpu.VMEM((CHUNK, DPAD), jnp.float32),
            pltpu.SemaphoreType.DMA,
        ],
    )
    u_rows, m_rows = gather(users, movies, up, mp)

    bm = 1024
    dot = pl.pallas_call(
        _dot_body,
        grid=(BATCH // bm,),
        in_specs=[pl.BlockSpec((bm, DPAD), lambda i: (i, 0)),
                  pl.BlockSpec((bm, DPAD), lambda i: (i, 0))],
        out_specs=pl.BlockSpec((bm,), lambda i: (i,)),
        out_shape=jax.ShapeDtypeStruct((BATCH,), jnp.float32),
    )
    return dot(u_rows, m_rows)


def kernel(cats, conts, u_weight, m_weight):
    del conts
    return _embedding_dot(cats.astype(jnp.int32), u_weight, m_weight)


# speedup vs baseline: 3.6570x; 2.8262x over previous
import jax
import jax.numpy as jnp
from jax import lax
from jax.experimental import pallas as pl
from jax.experimental.pallas import tpu as pltpu
from jax.experimental.pallas import tpu_sc as plsc

N_FACTORS = 50
BATCH = 16384
NC, NS, L = 2, 16, 16
NW = NC * NS
BPW = BATCH // NW
NG = BPW // L
C = 256
NGH = C // L


def _body(iu_hbm, im_hbm, u_hbm, m_hbm, o_hbm,
          iu_v, im_v, u_rows, m_rows, out_v, sem):
    wid = lax.axis_index("s") * NC + lax.axis_index("c")
    base = wid * BPW

    pltpu.sync_copy(iu_hbm.at[pl.ds(base, BPW)], iu_v)
    pltpu.sync_copy(im_hbm.at[pl.ds(base, BPW)], im_v)

    lanes = lax.iota(jnp.int32, L)

    def half(h, _):
        hb = h * C

        def issue(g, _):
            vu = iu_v[pl.ds(hb + g * L, L)]
            vm = im_v[pl.ds(hb + g * L, L)]
            for j in range(L):
                i = g * L + j
                pltpu.async_copy(u_hbm.at[pl.ds(vu[j], 1), :],
                                 u_rows.at[pl.ds(i, 1), :], sem)
                pltpu.async_copy(m_hbm.at[pl.ds(vm[j], 1), :],
                                 m_rows.at[pl.ds(i, 1), :], sem)
            return 0

        lax.fori_loop(0, NGH, issue, 0)

        def drain(i, _):
            pltpu.make_async_copy(u_hbm.at[pl.ds(0, 1), :],
                                  u_rows.at[pl.ds(0, 1), :], sem).wait()
            pltpu.make_async_copy(m_hbm.at[pl.ds(0, 1), :],
                                  m_rows.at[pl.ds(0, 1), :], sem).wait()
            return 0

        lax.fori_loop(0, C, drain, 0)

        def dot(g, _):
            rows = g * L + lanes
            acc = (plsc.load_gather(u_rows, [rows, jnp.zeros((L,), jnp.int32)])
                   * plsc.load_gather(m_rows, [rows, jnp.zeros((L,), jnp.int32)]))
            for f in range(1, N_FACTORS):
                cols = jnp.full((L,), f, jnp.int32)
                acc = acc + (plsc.load_gather(u_rows, [rows, cols])
                             * plsc.load_gather(m_rows, [rows, cols]))
            out_v[pl.ds(hb + g * L, L)] = acc
            return 0

        lax.fori_loop(0, NGH, dot, 0)
        return 0

    lax.fori_loop(0, BPW // C, half, 0)

    pltpu.sync_copy(out_v, o_hbm.at[pl.ds(base, BPW)])


@jax.jit
def _embedding_dot(cats, u_weight, m_weight):
    users = cats[:, 0]
    movies = cats[:, 1]
    mesh = plsc.VectorSubcoreMesh(core_axis_name="c", subcore_axis_name="s")
    run = pl.kernel(
        _body, mesh=mesh,
        compiler_params=pltpu.CompilerParams(needs_layout_passes=False),
        out_type=jax.ShapeDtypeStruct((BATCH,), jnp.float32),
        scratch_types=[
            pltpu.VMEM((BPW,), jnp.int32),
            pltpu.VMEM((BPW,), jnp.int32),
            pltpu.VMEM((C, N_FACTORS), jnp.float32),
            pltpu.VMEM((C, N_FACTORS), jnp.float32),
            pltpu.VMEM((BPW,), jnp.float32),
            pltpu.SemaphoreType.DMA,
        ],
    )
    return run(users, movies, u_weight, m_weight)


def kernel(cats, conts, u_weight, m_weight):
    del conts
    return _embedding_dot(cats.astype(jnp.int32), u_weight, m_weight)
